# Initial kernel scaffold; baseline (speedup 1.0000x reference)
import jax
import jax.numpy as jnp
from jax.experimental import pallas as pl


def kernel(x, W, bias):
    raise NotImplementedError("write your pallas kernel here")


# speedup vs baseline: 16.7626x; 16.7626x over previous
import functools
import jax
import jax.numpy as jnp
from jax import lax
from jax.experimental import pallas as pl
from jax.experimental.pallas import tpu as pltpu

C = 768
N = 1024
K = 32
NT_PROJ = 256
T_AGG = 128


def _featpos_body(x_ref, w_ref, b_ref, feat_ref, pos_ref):
    xb = x_ref[0]
    w = w_ref[...]
    fp = lax.dot_general(w, xb, (((1,), (0,)), ((), ())),
                         preferred_element_type=jnp.float32)
    fp = fp + b_ref[...]
    feat = fp[:C, :]
    posu = fp[C:, :]
    ss = jnp.sum(posu * posu, axis=0, keepdims=True)
    inv = 1.0 / jnp.clip(jnp.sqrt(ss), 1e-12)
    feat_ref[0] = feat
    pos_ref[0] = posu * inv


def _sortable_u32(x):
    ub = lax.bitcast_convert_type(x, jnp.uint32)
    neg = (ub >> 31).astype(jnp.bool_)
    return jnp.where(neg, ~ub, ub | jnp.uint32(0x80000000))


def _agg_body(pos_t_ref, pos_ref, feat_ref, out_ref):
    pos_t = pos_t_ref[0]
    pos_b = pos_ref[0]
    feat_b = feat_ref[0]
    sim = lax.dot_general(pos_t, pos_b, (((0,), (0,)), ((), ())),
                          preferred_element_type=jnp.float32)
    u = _sortable_u32(sim)

    def bit_step(i, cur):
        bit = jnp.uint32(1) << (jnp.uint32(31) - i.astype(jnp.uint32))
        t = cur | bit
        cnt = jnp.sum((u >= t).astype(jnp.float32), axis=1, keepdims=True)
        return jnp.where(cnt >= K, t, cur)

    cur = lax.fori_loop(0, 32, bit_step, jnp.zeros((T_AGG, 1), jnp.uint32))
    mask = u >= cur
    e = jnp.where(mask, jnp.exp(sim - 1.0), 0.0)
    s = jnp.sum(e, axis=1, keepdims=True)
    attn = e / s
    out = lax.dot_general(feat_b, attn, (((1,), (1,)), ((), ())),
                          preferred_element_type=jnp.float32)
    out_ref[0] = out


@jax.jit
def kernel(x, W, bias):
    b, c, h, w = x.shape
    n = h * w
    xr = x.reshape(b, c, n)
    brow = bias.reshape(2 * c, 1)

    feat, pos = pl.pallas_call(
        _featpos_body,
        grid=(b, n // NT_PROJ),
        in_specs=[
            pl.BlockSpec((1, c, NT_PROJ), lambda i, j: (i, 0, j)),
            pl.BlockSpec((2 * c, c), lambda i, j: (0, 0)),
            pl.BlockSpec((2 * c, 1), lambda i, j: (0, 0)),
        ],
        out_specs=[
            pl.BlockSpec((1, c, NT_PROJ), lambda i, j: (i, 0, j)),
            pl.BlockSpec((1, c, NT_PROJ), lambda i, j: (i, 0, j)),
        ],
        out_shape=[
            jax.ShapeDtypeStruct((b, c, n), jnp.float32),
            jax.ShapeDtypeStruct((b, c, n), jnp.float32),
        ],
    )(xr, W, brow)

    out = pl.pallas_call(
        _agg_body,
        grid=(b, n // T_AGG),
        in_specs=[
            pl.BlockSpec((1, c, T_AGG), lambda i, j: (i, 0, j)),
            pl.BlockSpec((1, c, n), lambda i, j: (i, 0, 0)),
            pl.BlockSpec((1, c, n), lambda i, j: (i, 0, 0)),
        ],
        out_specs=pl.BlockSpec((1, c, T_AGG), lambda i, j: (i, 0, j)),
        out_shape=jax.ShapeDtypeStruct((b, c, n), jnp.float32),
    )(pos, pos, feat)

    return out.reshape(b, c, h, w)
